# Initial kernel scaffold; baseline (speedup 1.0000x reference)
import jax
import jax.numpy as jnp
from jax.experimental import pallas as pl


def kernel(x, edge_index, W1, W2, lin1_w, lin1_b, bn_g, bn_b, lin3_w, lin3_b):
    raise NotImplementedError("write your pallas kernel here")


# speedup vs baseline: 46.0495x; 46.0495x over previous
import jax
import jax.numpy as jnp
from jax import lax
from jax.experimental import pallas as pl
from jax.experimental.pallas import tpu as pltpu
from jax.experimental.pallas import tpu_sc as plsc

B = 256
NPG = 111
EPG = NPG * 16
D1 = 111
D2 = 128
NHID = 256
K1 = 56
K2 = 28
AP = 112
AFLAT = NPG * AP

_NW = 32
_GPW = B // _NW


def _adj_body(src_hbm, dst_hbm, out_hbm, src_v, dst_v, acc_v):
  wid = lax.axis_index("s") * 2 + lax.axis_index("c")

  def per_graph(i, carry):
    g = wid * _GPW + i

    def zero(t, c2):
      for u in range(7):
        acc_v[pl.ds((t * 7 + u) * 16, 16)] = jnp.zeros((16,), jnp.float32)
      return c2

    lax.fori_loop(0, AFLAT // (16 * 7), zero, carry)

    pltpu.sync_copy(src_hbm.at[g], src_v)
    pltpu.sync_copy(dst_hbm.at[g], dst_v)

    def edges(j, c2):
      sl = src_v[pl.ds(j * 16, 16)]
      dl = dst_v[pl.ds(j * 16, 16)]
      flat = dl * AP + sl
      plsc.addupdate_scatter(acc_v, [flat], jnp.ones((16,), jnp.float32))
      return c2

    lax.fori_loop(0, EPG // 16, edges, carry)
    pltpu.sync_copy(acc_v, out_hbm.at[g])
    return carry

  lax.fori_loop(0, _GPW, per_graph, 0)


def _topk_select(score, n, k):
  f32 = jnp.float32
  score_row = jnp.transpose(score)
  ii = lax.broadcasted_iota(jnp.int32, (n, n), 0)
  jj = lax.broadcasted_iota(jnp.int32, (n, n), 1)
  beats = (score_row > score) | ((score_row == score) & (jj < ii))
  rank = jnp.sum(beats.astype(f32), axis=1, keepdims=True)
  maskf = (rank < float(k)).astype(f32)
  le = (jj <= ii).astype(f32)
  npos = (
      jnp.dot(le, maskf, preferred_element_type=f32, precision=lax.Precision.HIGHEST) - 1.0
  )
  pp = lax.broadcasted_iota(jnp.int32, (n, k), 1)
  npos_i = npos.astype(jnp.int32)
  return ((npos_i == pp) & (maskf > 0.0)).astype(f32)


def _gnn_body(x_ref, a_ref, w1_ref, w2_ref, h3_ref, x3_ref):
  f32 = jnp.float32
  x = x_ref[0]
  A = a_ref[0][:, :NPG]
  W1 = w1_ref[0]
  W2 = w2_ref[0]

  h = jnp.dot(x, W1, preferred_element_type=f32)

  deg = jnp.sum(A, axis=1, keepdims=True) + 1.0
  dinv = lax.rsqrt(deg)
  hs = dinv * h
  u = jnp.dot(A, hs, preferred_element_type=f32, precision=lax.Precision.HIGHEST) + hs
  h1 = jnp.maximum(dinv * u, 0.0)

  degp = jnp.sum(A, axis=1, keepdims=True) + 1e-10
  agg = jnp.dot(A, h1, preferred_element_type=f32, precision=lax.Precision.HIGHEST) / degp
  score = jnp.sum(jnp.abs(h1 - agg), axis=1, keepdims=True)

  S1T = _topk_select(score, NPG, K1)
  hk = lax.dot_general(
      S1T, h1, (((0,), (0,)), ((), ())), preferred_element_type=f32, precision=lax.Precision.HIGHEST
  )
  AS = jnp.dot(A, S1T, preferred_element_type=f32, precision=lax.Precision.HIGHEST)
  A2 = lax.dot_general(
      S1T, AS, (((0,), (0,)), ((), ())), preferred_element_type=f32, precision=lax.Precision.HIGHEST
  )

  x1 = jnp.concatenate(
      [
          jnp.max(hk, axis=0, keepdims=True),
          jnp.mean(hk, axis=0, keepdims=True),
      ],
      axis=1,
  )

  g2 = jnp.dot(hk, W2, preferred_element_type=f32)
  deg2 = jnp.sum(A2, axis=1, keepdims=True) + 1.0
  dinv2 = lax.rsqrt(deg2)
  gs = dinv2 * g2
  u2 = jnp.dot(A2, gs, preferred_element_type=f32, precision=lax.Precision.HIGHEST) + gs
  h2 = jnp.maximum(dinv2 * u2, 0.0)

  degp2 = jnp.sum(A2, axis=1, keepdims=True) + 1e-10
  agg2 = jnp.dot(A2, h2, preferred_element_type=f32, precision=lax.Precision.HIGHEST) / degp2
  score2 = jnp.sum(jnp.abs(h2 - agg2), axis=1, keepdims=True)

  S2T = _topk_select(score2, K1, K2)
  h3 = lax.dot_general(
      S2T, h2, (((0,), (0,)), ((), ())), preferred_element_type=f32, precision=lax.Precision.HIGHEST
  )

  x2 = jnp.concatenate(
      [
          jnp.max(h3, axis=0, keepdims=True),
          jnp.mean(h3, axis=0, keepdims=True),
      ],
      axis=1,
  )

  h3_ref[0] = h3
  x3_ref[0] = jnp.maximum(x1, 0.0) + jnp.maximum(x2, 0.0)


def _head_body(
    xf_ref, x3_ref, w1a_ref, w1b_ref, b1_ref, g_ref, bb_ref, w3_ref, b3_ref,
    out_ref,
):
  f32 = jnp.float32
  xf = jnp.maximum(xf_ref[...], 0.0)
  xc = jnp.concatenate([xf, x3_ref[...]], axis=1)
  w1 = jnp.concatenate([w1a_ref[...], w1b_ref[...]], axis=0)
  pre = jnp.dot(xc, w1, preferred_element_type=f32) + b1_ref[...]
  feats = jnp.maximum(pre, 0.0)
  mu = jnp.mean(feats, axis=0, keepdims=True)
  var = jnp.mean((feats - mu) ** 2, axis=0, keepdims=True)
  normed = (feats - mu) * lax.rsqrt(var + 1e-5) * g_ref[...] + bb_ref[...]
  out_ref[...] = (
      jnp.dot(normed, w3_ref[...], preferred_element_type=f32) + b3_ref[...]
  )


def kernel(x, edge_index, W1, W2, lin1_w, lin1_b, bn_g, bn_b, lin3_w, lin3_b):
  off = (jnp.arange(B, dtype=jnp.int32) * NPG)[:, None]
  srcl = edge_index[0].reshape(B, EPG) - off
  dstl = edge_index[1].reshape(B, EPG) - off

  aflat = pl.kernel(
      _adj_body,
      out_type=jax.ShapeDtypeStruct((B, AFLAT), jnp.float32),
      mesh=plsc.VectorSubcoreMesh(core_axis_name="c", subcore_axis_name="s"),
      scratch_types=[
          pltpu.VMEM((EPG,), jnp.int32),
          pltpu.VMEM((EPG,), jnp.int32),
          pltpu.VMEM((AFLAT,), jnp.float32),
      ],
      compiler_params=pltpu.CompilerParams(needs_layout_passes=False),
  )(srcl, dstl)
  A = aflat.reshape(B, NPG, AP)

  xg = x.reshape(B, NPG, D1)
  h3, x3 = pl.pallas_call(
      _gnn_body,
      grid=(B,),
      in_specs=[
          pl.BlockSpec((1, NPG, D1), lambda i: (i, 0, 0)),
          pl.BlockSpec((1, NPG, AP), lambda i: (i, 0, 0)),
          pl.BlockSpec((1, D1, D2), lambda i: (0, 0, 0)),
          pl.BlockSpec((1, D2, D2), lambda i: (0, 0, 0)),
      ],
      out_specs=[
          pl.BlockSpec((1, K2, D2), lambda i: (i, 0, 0)),
          pl.BlockSpec((1, 1, NHID), lambda i: (i, 0, 0)),
      ],
      out_shape=[
          jax.ShapeDtypeStruct((B, K2, D2), jnp.float32),
          jax.ShapeDtypeStruct((B, 1, NHID), jnp.float32),
      ],
      compiler_params=pltpu.CompilerParams(
          dimension_semantics=("arbitrary",)
      ),
  )(xg, A, W1.reshape(1, D1, D2), W2.reshape(1, D2, D2))

  xf = h3.reshape(B, K2 * D2)
  x3 = x3.reshape(B, NHID)

  out = pl.pallas_call(
      _head_body,
      out_shape=jax.ShapeDtypeStruct((B, 128), jnp.float32),
  )(
      xf,
      x3,
      lin1_w[: K2 * D2],
      lin1_w[K2 * D2 :],
      lin1_b.reshape(1, NHID),
      bn_g.reshape(1, NHID),
      bn_b.reshape(1, NHID),
      jnp.pad(lin3_w, ((0, 0), (0, 127))),
      jnp.pad(lin3_b.reshape(1, 1), ((0, 0), (0, 127))),
  )
  return out[:, 0]


# speedup vs baseline: 51.0892x; 1.1094x over previous
import jax
import jax.numpy as jnp
from jax import lax
from jax.experimental import pallas as pl
from jax.experimental.pallas import tpu as pltpu
from jax.experimental.pallas import tpu_sc as plsc

B = 256
NPG = 111
EPG = NPG * 16
D1 = 111
D2 = 128
NHID = 256
K1 = 56
K2 = 28
AP = 112
AFLAT = NPG * AP

_NW = 32
_GPW = B // _NW


def _adj_body(src_hbm, dst_hbm, out_hbm, src_v, dst_v, acc_v):
  wid = lax.axis_index("s") * 2 + lax.axis_index("c")

  def per_graph(i, carry):
    g = wid * _GPW + i

    def zero(t, c2):
      for u in range(7):
        acc_v[pl.ds((t * 7 + u) * 16, 16)] = jnp.zeros((16,), jnp.float32)
      return c2

    lax.fori_loop(0, AFLAT // (16 * 7), zero, carry)

    pltpu.sync_copy(src_hbm.at[g], src_v)
    pltpu.sync_copy(dst_hbm.at[g], dst_v)

    def edges(j, c2):
      sl = src_v[pl.ds(j * 16, 16)]
      dl = dst_v[pl.ds(j * 16, 16)]
      flat = dl * AP + sl
      plsc.addupdate_scatter(acc_v, [flat], jnp.ones((16,), jnp.float32))
      return c2

    lax.fori_loop(0, EPG // 16, edges, carry)
    pltpu.sync_copy(acc_v, out_hbm.at[g])
    return carry

  lax.fori_loop(0, _GPW, per_graph, 0)


def _topk_select(score, n, k):
  f32 = jnp.float32
  score_row = jnp.transpose(score)
  ii = lax.broadcasted_iota(jnp.int32, (n, n), 0)
  jj = lax.broadcasted_iota(jnp.int32, (n, n), 1)
  beats = (score_row > score) | ((score_row == score) & (jj < ii))
  rank = jnp.sum(beats.astype(f32), axis=1, keepdims=True)
  maskf = (rank < float(k)).astype(f32)
  le = (jj <= ii).astype(f32)
  npos = (
      jnp.dot(le, maskf, preferred_element_type=f32, precision=lax.Precision.HIGHEST) - 1.0
  )
  pp = lax.broadcasted_iota(jnp.int32, (n, k), 1)
  npos_i = npos.astype(jnp.int32)
  return ((npos_i == pp) & (maskf > 0.0)).astype(f32)


GPP = 4


def _gnn_body(x_ref, a_ref, w1_ref, w2_ref, h3_ref, x3_ref):
  W1 = w1_ref[0]
  W2 = w2_ref[0]
  for g in range(GPP):
    h3, x3 = _gnn_one(x_ref[g], a_ref[g][:, :NPG], W1, W2)
    h3_ref[g] = h3
    x3_ref[g] = x3


def _gnn_one(x, A, W1, W2):
  f32 = jnp.float32

  h = jnp.dot(x, W1, preferred_element_type=f32)

  deg = jnp.sum(A, axis=1, keepdims=True) + 1.0
  dinv = lax.rsqrt(deg)
  hs = dinv * h
  u = jnp.dot(A, hs, preferred_element_type=f32, precision=lax.Precision.HIGHEST) + hs
  h1 = jnp.maximum(dinv * u, 0.0)

  degp = jnp.sum(A, axis=1, keepdims=True) + 1e-10
  agg = jnp.dot(A, h1, preferred_element_type=f32, precision=lax.Precision.HIGHEST) / degp
  score = jnp.sum(jnp.abs(h1 - agg), axis=1, keepdims=True)

  S1T = _topk_select(score, NPG, K1)
  hk = lax.dot_general(
      S1T, h1, (((0,), (0,)), ((), ())), preferred_element_type=f32, precision=lax.Precision.HIGHEST
  )
  AS = jnp.dot(A, S1T, preferred_element_type=f32, precision=lax.Precision.HIGHEST)
  A2 = lax.dot_general(
      S1T, AS, (((0,), (0,)), ((), ())), preferred_element_type=f32, precision=lax.Precision.HIGHEST
  )

  x1 = jnp.concatenate(
      [
          jnp.max(hk, axis=0, keepdims=True),
          jnp.mean(hk, axis=0, keepdims=True),
      ],
      axis=1,
  )

  g2 = jnp.dot(hk, W2, preferred_element_type=f32)
  deg2 = jnp.sum(A2, axis=1, keepdims=True) + 1.0
  dinv2 = lax.rsqrt(deg2)
  gs = dinv2 * g2
  u2 = jnp.dot(A2, gs, preferred_element_type=f32, precision=lax.Precision.HIGHEST) + gs
  h2 = jnp.maximum(dinv2 * u2, 0.0)

  degp2 = jnp.sum(A2, axis=1, keepdims=True) + 1e-10
  agg2 = jnp.dot(A2, h2, preferred_element_type=f32, precision=lax.Precision.HIGHEST) / degp2
  score2 = jnp.sum(jnp.abs(h2 - agg2), axis=1, keepdims=True)

  S2T = _topk_select(score2, K1, K2)
  h3 = lax.dot_general(
      S2T, h2, (((0,), (0,)), ((), ())), preferred_element_type=f32, precision=lax.Precision.HIGHEST
  )

  x2 = jnp.concatenate(
      [
          jnp.max(h3, axis=0, keepdims=True),
          jnp.mean(h3, axis=0, keepdims=True),
      ],
      axis=1,
  )

  return h3, jnp.maximum(x1, 0.0) + jnp.maximum(x2, 0.0)


def _head_body(
    xf_ref, x3_ref, w1a_ref, w1b_ref, b1_ref, g_ref, bb_ref, w3_ref, b3_ref,
    out_ref,
):
  f32 = jnp.float32
  xf = jnp.maximum(xf_ref[...], 0.0)
  xc = jnp.concatenate([xf, x3_ref[...]], axis=1)
  w1 = jnp.concatenate([w1a_ref[...], w1b_ref[...]], axis=0)
  pre = jnp.dot(xc, w1, preferred_element_type=f32) + b1_ref[...]
  feats = jnp.maximum(pre, 0.0)
  mu = jnp.mean(feats, axis=0, keepdims=True)
  var = jnp.mean((feats - mu) ** 2, axis=0, keepdims=True)
  normed = (feats - mu) * lax.rsqrt(var + 1e-5) * g_ref[...] + bb_ref[...]
  out_ref[...] = (
      jnp.dot(normed, w3_ref[...], preferred_element_type=f32) + b3_ref[...]
  )


def kernel(x, edge_index, W1, W2, lin1_w, lin1_b, bn_g, bn_b, lin3_w, lin3_b):
  off = (jnp.arange(B, dtype=jnp.int32) * NPG)[:, None]
  srcl = edge_index[0].reshape(B, EPG) - off
  dstl = edge_index[1].reshape(B, EPG) - off

  aflat = pl.kernel(
      _adj_body,
      out_type=jax.ShapeDtypeStruct((B, AFLAT), jnp.float32),
      mesh=plsc.VectorSubcoreMesh(core_axis_name="c", subcore_axis_name="s"),
      scratch_types=[
          pltpu.VMEM((EPG,), jnp.int32),
          pltpu.VMEM((EPG,), jnp.int32),
          pltpu.VMEM((AFLAT,), jnp.float32),
      ],
      compiler_params=pltpu.CompilerParams(needs_layout_passes=False),
  )(srcl, dstl)
  A = aflat.reshape(B, NPG, AP)

  xg = x.reshape(B, NPG, D1)
  h3, x3 = pl.pallas_call(
      _gnn_body,
      grid=(B // GPP,),
      in_specs=[
          pl.BlockSpec((GPP, NPG, D1), lambda i: (i, 0, 0)),
          pl.BlockSpec((GPP, NPG, AP), lambda i: (i, 0, 0)),
          pl.BlockSpec((1, D1, D2), lambda i: (0, 0, 0)),
          pl.BlockSpec((1, D2, D2), lambda i: (0, 0, 0)),
      ],
      out_specs=[
          pl.BlockSpec((GPP, K2, D2), lambda i: (i, 0, 0)),
          pl.BlockSpec((GPP, 1, NHID), lambda i: (i, 0, 0)),
      ],
      out_shape=[
          jax.ShapeDtypeStruct((B, K2, D2), jnp.float32),
          jax.ShapeDtypeStruct((B, 1, NHID), jnp.float32),
      ],
      compiler_params=pltpu.CompilerParams(
          dimension_semantics=("arbitrary",)
      ),
  )(xg, A, W1.reshape(1, D1, D2), W2.reshape(1, D2, D2))

  xf = h3.reshape(B, K2 * D2)
  x3 = x3.reshape(B, NHID)

  out = pl.pallas_call(
      _head_body,
      out_shape=jax.ShapeDtypeStruct((B, 128), jnp.float32),
  )(
      xf,
      x3,
      lin1_w[: K2 * D2],
      lin1_w[K2 * D2 :],
      lin1_b.reshape(1, NHID),
      bn_g.reshape(1, NHID),
      bn_b.reshape(1, NHID),
      jnp.pad(lin3_w, ((0, 0), (0, 127))),
      jnp.pad(lin3_b.reshape(1, 1), ((0, 0), (0, 127))),
  )
  return out[:, 0]


# speedup vs baseline: 52.0685x; 1.0192x over previous
import jax
import jax.numpy as jnp
from jax import lax
from jax.experimental import pallas as pl
from jax.experimental.pallas import tpu as pltpu
from jax.experimental.pallas import tpu_sc as plsc

B = 256
NPG = 111
EPG = NPG * 16
D1 = 111
D2 = 128
NHID = 256
K1 = 56
K2 = 28
AP = 112
AFLAT = NPG * AP

_NW = 32
_GPW = B // _NW


def _adj_body(src_hbm, dst_hbm, out_hbm, src_v, dst_v, acc_v):
  wid = lax.axis_index("s") * 2 + lax.axis_index("c")

  def per_graph(i, carry):
    g = wid * _GPW + i

    def zero(t, c2):
      for u in range(7):
        acc_v[pl.ds((t * 7 + u) * 16, 16)] = jnp.zeros((16,), jnp.float32)
      return c2

    lax.fori_loop(0, AFLAT // (16 * 7), zero, carry)

    pltpu.sync_copy(src_hbm.at[g], src_v)
    pltpu.sync_copy(dst_hbm.at[g], dst_v)

    def edges(j, c2):
      sl = src_v[pl.ds(j * 16, 16)]
      dl = dst_v[pl.ds(j * 16, 16)]
      flat = dl * AP + sl
      plsc.addupdate_scatter(acc_v, [flat], jnp.ones((16,), jnp.float32))
      return c2

    lax.fori_loop(0, EPG // 16, edges, carry)
    pltpu.sync_copy(acc_v, out_hbm.at[g])
    return carry

  lax.fori_loop(0, _GPW, per_graph, 0)


def _topk_select(score, n, k):
  f32 = jnp.float32
  score_row = jnp.transpose(score)
  ii = lax.broadcasted_iota(jnp.int32, (n, n), 0)
  jj = lax.broadcasted_iota(jnp.int32, (n, n), 1)
  beats = (score_row > score) | ((score_row == score) & (jj < ii))
  rank = jnp.sum(beats.astype(f32), axis=1, keepdims=True)
  maskf = (rank < float(k)).astype(f32)
  le = (jj <= ii).astype(f32)
  npos = (
      jnp.dot(le, maskf, preferred_element_type=f32, precision=lax.Precision.HIGHEST) - 1.0
  )
  pp = lax.broadcasted_iota(jnp.int32, (n, k), 1)
  npos_i = npos.astype(jnp.int32)
  return ((npos_i == pp) & (maskf > 0.0)).astype(f32)


GPP = 8


def _gnn_body(x_ref, a_ref, w1_ref, w2_ref, h3_ref, x3_ref):
  W1 = w1_ref[0]
  W2 = w2_ref[0]
  for g in range(GPP):
    h3, x3 = _gnn_one(x_ref[g], a_ref[g][:, :NPG], W1, W2)
    h3_ref[g] = h3
    x3_ref[g] = x3


def _gnn_one(x, A, W1, W2):
  f32 = jnp.float32

  h = jnp.dot(x, W1, preferred_element_type=f32)

  deg = jnp.sum(A, axis=1, keepdims=True) + 1.0
  dinv = lax.rsqrt(deg)
  hs = dinv * h
  u = jnp.dot(A, hs, preferred_element_type=f32, precision=lax.Precision.HIGHEST) + hs
  h1 = jnp.maximum(dinv * u, 0.0)

  degp = jnp.sum(A, axis=1, keepdims=True) + 1e-10
  agg = jnp.dot(A, h1, preferred_element_type=f32, precision=lax.Precision.HIGHEST) / degp
  score = jnp.sum(jnp.abs(h1 - agg), axis=1, keepdims=True)

  S1T = _topk_select(score, NPG, K1)
  hk = lax.dot_general(
      S1T, h1, (((0,), (0,)), ((), ())), preferred_element_type=f32, precision=lax.Precision.HIGHEST
  )
  AS = jnp.dot(A, S1T, preferred_element_type=f32, precision=lax.Precision.HIGHEST)
  A2 = lax.dot_general(
      S1T, AS, (((0,), (0,)), ((), ())), preferred_element_type=f32, precision=lax.Precision.HIGHEST
  )

  x1 = jnp.concatenate(
      [
          jnp.max(hk, axis=0, keepdims=True),
          jnp.mean(hk, axis=0, keepdims=True),
      ],
      axis=1,
  )

  g2 = jnp.dot(hk, W2, preferred_element_type=f32)
  deg2 = jnp.sum(A2, axis=1, keepdims=True) + 1.0
  dinv2 = lax.rsqrt(deg2)
  gs = dinv2 * g2
  u2 = jnp.dot(A2, gs, preferred_element_type=f32, precision=lax.Precision.HIGHEST) + gs
  h2 = jnp.maximum(dinv2 * u2, 0.0)

  degp2 = jnp.sum(A2, axis=1, keepdims=True) + 1e-10
  agg2 = jnp.dot(A2, h2, preferred_element_type=f32, precision=lax.Precision.HIGHEST) / degp2
  score2 = jnp.sum(jnp.abs(h2 - agg2), axis=1, keepdims=True)

  S2T = _topk_select(score2, K1, K2)
  h3 = lax.dot_general(
      S2T, h2, (((0,), (0,)), ((), ())), preferred_element_type=f32, precision=lax.Precision.HIGHEST
  )

  x2 = jnp.concatenate(
      [
          jnp.max(h3, axis=0, keepdims=True),
          jnp.mean(h3, axis=0, keepdims=True),
      ],
      axis=1,
  )

  return h3, jnp.maximum(x1, 0.0) + jnp.maximum(x2, 0.0)


def _head_body(
    xf_ref, x3_ref, w1a_ref, w1b_ref, b1_ref, g_ref, bb_ref, w3_ref, b3_ref,
    out_ref,
):
  f32 = jnp.float32
  xf = jnp.maximum(xf_ref[...], 0.0)
  xc = jnp.concatenate([xf, x3_ref[...]], axis=1)
  w1 = jnp.concatenate([w1a_ref[...], w1b_ref[...]], axis=0)
  pre = jnp.dot(xc, w1, preferred_element_type=f32) + b1_ref[...]
  feats = jnp.maximum(pre, 0.0)
  mu = jnp.mean(feats, axis=0, keepdims=True)
  var = jnp.mean((feats - mu) ** 2, axis=0, keepdims=True)
  normed = (feats - mu) * lax.rsqrt(var + 1e-5) * g_ref[...] + bb_ref[...]
  out_ref[...] = (
      jnp.dot(normed, w3_ref[...], preferred_element_type=f32) + b3_ref[...]
  )


def kernel(x, edge_index, W1, W2, lin1_w, lin1_b, bn_g, bn_b, lin3_w, lin3_b):
  off = (jnp.arange(B, dtype=jnp.int32) * NPG)[:, None]
  srcl = edge_index[0].reshape(B, EPG) - off
  dstl = edge_index[1].reshape(B, EPG) - off

  aflat = pl.kernel(
      _adj_body,
      out_type=jax.ShapeDtypeStruct((B, AFLAT), jnp.float32),
      mesh=plsc.VectorSubcoreMesh(core_axis_name="c", subcore_axis_name="s"),
      scratch_types=[
          pltpu.VMEM((EPG,), jnp.int32),
          pltpu.VMEM((EPG,), jnp.int32),
          pltpu.VMEM((AFLAT,), jnp.float32),
      ],
      compiler_params=pltpu.CompilerParams(needs_layout_passes=False),
  )(srcl, dstl)
  A = aflat.reshape(B, NPG, AP)

  xg = x.reshape(B, NPG, D1)
  h3, x3 = pl.pallas_call(
      _gnn_body,
      grid=(B // GPP,),
      in_specs=[
          pl.BlockSpec((GPP, NPG, D1), lambda i: (i, 0, 0)),
          pl.BlockSpec((GPP, NPG, AP), lambda i: (i, 0, 0)),
          pl.BlockSpec((1, D1, D2), lambda i: (0, 0, 0)),
          pl.BlockSpec((1, D2, D2), lambda i: (0, 0, 0)),
      ],
      out_specs=[
          pl.BlockSpec((GPP, K2, D2), lambda i: (i, 0, 0)),
          pl.BlockSpec((GPP, 1, NHID), lambda i: (i, 0, 0)),
      ],
      out_shape=[
          jax.ShapeDtypeStruct((B, K2, D2), jnp.float32),
          jax.ShapeDtypeStruct((B, 1, NHID), jnp.float32),
      ],
      compiler_params=pltpu.CompilerParams(
          dimension_semantics=("arbitrary",)
      ),
  )(xg, A, W1.reshape(1, D1, D2), W2.reshape(1, D2, D2))

  xf = h3.reshape(B, K2 * D2)
  x3 = x3.reshape(B, NHID)

  out = pl.pallas_call(
      _head_body,
      out_shape=jax.ShapeDtypeStruct((B, 128), jnp.float32),
  )(
      xf,
      x3,
      lin1_w[: K2 * D2],
      lin1_w[K2 * D2 :],
      lin1_b.reshape(1, NHID),
      bn_g.reshape(1, NHID),
      bn_b.reshape(1, NHID),
      jnp.pad(lin3_w, ((0, 0), (0, 127))),
      jnp.pad(lin3_b.reshape(1, 1), ((0, 0), (0, 127))),
  )
  return out[:, 0]


# speedup vs baseline: 56.7538x; 1.0900x over previous
import jax
import jax.numpy as jnp
from jax import lax
from jax.experimental import pallas as pl
from jax.experimental.pallas import tpu as pltpu
from jax.experimental.pallas import tpu_sc as plsc

B = 256
NPG = 111
EPG = NPG * 16
D1 = 111
D2 = 128
NHID = 256
K1 = 56
K2 = 28
AP = 112
AFLAT = NPG * AP

_NW = 32
_GPW = B // _NW


def _adj_body(src_hbm, dst_hbm, out_hbm, src_v, dst_v, acc_v):
  wid = lax.axis_index("s") * 2 + lax.axis_index("c")

  def per_graph(i, carry):
    g = wid * _GPW + i

    def zero(t, c2):
      for u in range(7):
        acc_v[t, pl.ds(u * 16, 16)] = jnp.zeros((16,), jnp.float32)
      return c2

    lax.fori_loop(0, NPG, zero, carry)

    pltpu.sync_copy(src_hbm.at[g], src_v)
    pltpu.sync_copy(dst_hbm.at[g], dst_v)

    def edges(j, c2):
      sl = src_v[pl.ds(j * 16, 16)]
      dl = dst_v[pl.ds(j * 16, 16)]
      plsc.addupdate_scatter(acc_v, [dl, sl], jnp.ones((16,), jnp.float32))
      return c2

    lax.fori_loop(0, EPG // 16, edges, carry)
    pltpu.sync_copy(acc_v, out_hbm.at[g])
    return carry

  lax.fori_loop(0, _GPW, per_graph, 0)


def _topk_select(score, n, k, lt, le):
  f32 = jnp.float32
  score_row = jnp.transpose(score)
  beats = (score_row > score) | ((score_row == score) & lt)
  rank = jnp.sum(beats.astype(f32), axis=1, keepdims=True)
  maskf = (rank < float(k)).astype(f32)
  npos = jnp.dot(le, maskf, preferred_element_type=f32) - 1.0
  pp = lax.broadcasted_iota(jnp.int32, (n, k), 1)
  npos_i = npos.astype(jnp.int32)
  return ((npos_i == pp) & (maskf > 0.0)).astype(f32)


GPP = 8


def _lt_mat(n):
  ii = lax.broadcasted_iota(jnp.int32, (n, n), 0)
  jj = lax.broadcasted_iota(jnp.int32, (n, n), 1)
  return jj < ii, (jj <= ii).astype(jnp.float32)


def _gnn_body(x_ref, a_ref, w1_ref, w2_ref, h3_ref, x3_ref):
  f32 = jnp.float32
  W1 = w1_ref[0]
  W2 = w2_ref[0]
  lt1 = _lt_mat(NPG)
  lt2 = _lt_mat(K1)
  hall = jnp.dot(
      x_ref[...].reshape(GPP * NPG, D1), W1, preferred_element_type=f32
  )
  for g in range(GPP):
    h3, x3 = _gnn_one(
        hall[g * NPG : (g + 1) * NPG], a_ref[g][:, :NPG], W2, lt1, lt2
    )
    h3_ref[g] = h3
    x3_ref[g] = x3


def _gnn_one(h, A, W2, lt1, lt2):
  f32 = jnp.float32

  deg = jnp.sum(A, axis=1, keepdims=True) + 1.0
  dinv = lax.rsqrt(deg)
  hs = dinv * h
  u = jnp.dot(A, hs, preferred_element_type=f32, precision=lax.Precision.HIGHEST) + hs
  h1 = jnp.maximum(dinv * u, 0.0)

  degp = jnp.sum(A, axis=1, keepdims=True) + 1e-10
  agg = jnp.dot(A, h1, preferred_element_type=f32, precision=lax.Precision.HIGHEST) / degp
  score = jnp.sum(jnp.abs(h1 - agg), axis=1, keepdims=True)

  S1T = _topk_select(score, NPG, K1, *lt1)
  hk = lax.dot_general(
      S1T, h1, (((0,), (0,)), ((), ())), preferred_element_type=f32, precision=lax.Precision.HIGHEST
  )
  AS = jnp.dot(A, S1T, preferred_element_type=f32, precision=lax.Precision.HIGHEST)
  A2 = lax.dot_general(
      S1T, AS, (((0,), (0,)), ((), ())), preferred_element_type=f32, precision=lax.Precision.HIGHEST
  )

  x1 = jnp.concatenate(
      [
          jnp.max(hk, axis=0, keepdims=True),
          jnp.mean(hk, axis=0, keepdims=True),
      ],
      axis=1,
  )

  g2 = jnp.dot(hk, W2, preferred_element_type=f32)
  deg2 = jnp.sum(A2, axis=1, keepdims=True) + 1.0
  dinv2 = lax.rsqrt(deg2)
  gs = dinv2 * g2
  u2 = jnp.dot(A2, gs, preferred_element_type=f32, precision=lax.Precision.HIGHEST) + gs
  h2 = jnp.maximum(dinv2 * u2, 0.0)

  degp2 = jnp.sum(A2, axis=1, keepdims=True) + 1e-10
  agg2 = jnp.dot(A2, h2, preferred_element_type=f32, precision=lax.Precision.HIGHEST) / degp2
  score2 = jnp.sum(jnp.abs(h2 - agg2), axis=1, keepdims=True)

  S2T = _topk_select(score2, K1, K2, *lt2)
  h3 = lax.dot_general(
      S2T, h2, (((0,), (0,)), ((), ())), preferred_element_type=f32, precision=lax.Precision.HIGHEST
  )

  x2 = jnp.concatenate(
      [
          jnp.max(h3, axis=0, keepdims=True),
          jnp.mean(h3, axis=0, keepdims=True),
      ],
      axis=1,
  )

  return h3, jnp.maximum(x1, 0.0) + jnp.maximum(x2, 0.0)


def _head_body(
    xf_ref, x3_ref, w1a_ref, w1b_ref, b1_ref, g_ref, bb_ref, w3_ref, b3_ref,
    out_ref,
):
  f32 = jnp.float32
  xf = jnp.maximum(xf_ref[...], 0.0)
  xc = jnp.concatenate([xf, x3_ref[...]], axis=1)
  w1 = jnp.concatenate([w1a_ref[...], w1b_ref[...]], axis=0)
  pre = jnp.dot(xc, w1, preferred_element_type=f32) + b1_ref[...]
  feats = jnp.maximum(pre, 0.0)
  mu = jnp.mean(feats, axis=0, keepdims=True)
  var = jnp.mean((feats - mu) ** 2, axis=0, keepdims=True)
  normed = (feats - mu) * lax.rsqrt(var + 1e-5) * g_ref[...] + bb_ref[...]
  out_ref[...] = (
      jnp.dot(normed, w3_ref[...], preferred_element_type=f32) + b3_ref[...]
  )


def kernel(x, edge_index, W1, W2, lin1_w, lin1_b, bn_g, bn_b, lin3_w, lin3_b):
  off = (jnp.arange(B, dtype=jnp.int32) * NPG)[:, None]
  srcl = edge_index[0].reshape(B, EPG) - off
  dstl = edge_index[1].reshape(B, EPG) - off

  A = pl.kernel(
      _adj_body,
      out_type=jax.ShapeDtypeStruct((B, NPG, AP), jnp.float32),
      mesh=plsc.VectorSubcoreMesh(core_axis_name="c", subcore_axis_name="s"),
      scratch_types=[
          pltpu.VMEM((EPG,), jnp.int32),
          pltpu.VMEM((EPG,), jnp.int32),
          pltpu.VMEM((NPG, AP), jnp.float32),
      ],
      compiler_params=pltpu.CompilerParams(needs_layout_passes=False),
  )(srcl, dstl)

  xg = x.reshape(B, NPG, D1)
  h3, x3 = pl.pallas_call(
      _gnn_body,
      grid=(B // GPP,),
      in_specs=[
          pl.BlockSpec((GPP, NPG, D1), lambda i: (i, 0, 0)),
          pl.BlockSpec((GPP, NPG, AP), lambda i: (i, 0, 0)),
          pl.BlockSpec((1, D1, D2), lambda i: (0, 0, 0)),
          pl.BlockSpec((1, D2, D2), lambda i: (0, 0, 0)),
      ],
      out_specs=[
          pl.BlockSpec((GPP, K2, D2), lambda i: (i, 0, 0)),
          pl.BlockSpec((GPP, 1, NHID), lambda i: (i, 0, 0)),
      ],
      out_shape=[
          jax.ShapeDtypeStruct((B, K2, D2), jnp.float32),
          jax.ShapeDtypeStruct((B, 1, NHID), jnp.float32),
      ],
      compiler_params=pltpu.CompilerParams(
          dimension_semantics=("arbitrary",)
      ),
  )(xg, A, W1.reshape(1, D1, D2), W2.reshape(1, D2, D2))

  xf = h3.reshape(B, K2 * D2)
  x3 = x3.reshape(B, NHID)

  out = pl.pallas_call(
      _head_body,
      out_shape=jax.ShapeDtypeStruct((B, 128), jnp.float32),
  )(
      xf,
      x3,
      lin1_w[: K2 * D2],
      lin1_w[K2 * D2 :],
      lin1_b.reshape(1, NHID),
      bn_g.reshape(1, NHID),
      bn_b.reshape(1, NHID),
      jnp.pad(lin3_w, ((0, 0), (0, 127))),
      jnp.pad(lin3_b.reshape(1, 1), ((0, 0), (0, 127))),
  )
  return out[:, 0]


# speedup vs baseline: 67.2918x; 1.1857x over previous
import jax
import jax.numpy as jnp
from jax import lax
from jax.experimental import pallas as pl
from jax.experimental.pallas import tpu as pltpu
from jax.experimental.pallas import tpu_sc as plsc

B = 256
NPG = 111
EPG = NPG * 16
D1 = 111
D2 = 128
NHID = 256
K1 = 56
K2 = 28
AP = 112
AFLAT = NPG * AP

_NW = 32
_GPW = B // _NW


def _adj_body(src_hbm, dst_hbm, out_hbm, src_v, dst_v, acc_v):
  wid = lax.axis_index("s") * 2 + lax.axis_index("c")

  def per_graph(i, carry):
    g = wid * _GPW + i

    def zero(t, c2):
      for u in range(7):
        acc_v[t, pl.ds(u * 16, 16)] = jnp.zeros((16,), jnp.float32)
      return c2

    lax.fori_loop(0, NPG, zero, carry)

    pltpu.sync_copy(src_hbm.at[g], src_v)
    pltpu.sync_copy(dst_hbm.at[g], dst_v)

    def edges(j, c2):
      sl = src_v[pl.ds(j * 16, 16)]
      dl = dst_v[pl.ds(j * 16, 16)]
      plsc.addupdate_scatter(acc_v, [dl, sl], jnp.ones((16,), jnp.float32))
      return c2

    lax.fori_loop(0, EPG // 16, edges, carry)
    pltpu.sync_copy(acc_v, out_hbm.at[g])
    return carry

  lax.fori_loop(0, _GPW, per_graph, 0)


def _topk_select(score, n, k, lt, le):
  f32 = jnp.float32
  score_row = jnp.transpose(score)
  beats = (score_row > score) | ((score_row == score) & lt)
  rank = jnp.sum(beats.astype(f32), axis=1, keepdims=True)
  maskf = (rank < float(k)).astype(f32)
  npos = jnp.dot(le, maskf, preferred_element_type=f32) - 1.0
  pp = lax.broadcasted_iota(jnp.int32, (n, k), 1)
  npos_i = npos.astype(jnp.int32)
  return ((npos_i == pp) & (maskf > 0.0)).astype(f32)


GPP = 8


def _lt_mat(n):
  ii = lax.broadcasted_iota(jnp.int32, (n, n), 0)
  jj = lax.broadcasted_iota(jnp.int32, (n, n), 1)
  return jj < ii, (jj <= ii).astype(jnp.float32)


def _gnn_body(x_ref, a_ref, w1_ref, w2_ref, h3_ref, x3_ref):
  f32 = jnp.float32
  W1 = w1_ref[0]
  W2 = w2_ref[0]
  lt1 = _lt_mat(NPG)
  lt2 = _lt_mat(K1)
  hall = jnp.dot(
      x_ref[...].reshape(GPP * NPG, D1), W1, preferred_element_type=f32
  )
  hks, a2s, x1s = [], [], []
  for g in range(GPP):
    hk, A2, x1 = _stage1(hall[g * NPG : (g + 1) * NPG], a_ref[g][:, :NPG], lt1)
    hks.append(hk)
    a2s.append(A2)
    x1s.append(x1)
  g2all = jnp.dot(
      jnp.concatenate(hks, axis=0), W2, preferred_element_type=f32
  )
  for g in range(GPP):
    h3, x3 = _stage2(
        g2all[g * K1 : (g + 1) * K1], a2s[g], x1s[g], lt2
    )
    h3_ref[g] = h3
    x3_ref[g] = x3


def _stage1(h, A, lt1):
  f32 = jnp.float32
  hp = lax.Precision.HIGHEST

  rs = jnp.sum(A, axis=1, keepdims=True)
  dinv = lax.rsqrt(rs + 1.0)
  hs = dinv * h
  u = jnp.dot(A, hs, preferred_element_type=f32, precision=hp) + hs
  h1 = jnp.maximum(dinv * u, 0.0)

  agg = jnp.dot(A, h1, preferred_element_type=f32, precision=hp) / (
      rs + 1e-10
  )
  score = jnp.sum(jnp.abs(h1 - agg), axis=1, keepdims=True)

  S1T = _topk_select(score, NPG, K1, *lt1)
  AS = jnp.dot(A, S1T, preferred_element_type=f32, precision=hp)
  hkA2 = lax.dot_general(
      S1T,
      jnp.concatenate([h1, AS], axis=1),
      (((0,), (0,)), ((), ())),
      preferred_element_type=f32,
      precision=hp,
  )
  hk = hkA2[:, :D2]
  A2 = hkA2[:, D2:]

  x1 = jnp.concatenate(
      [
          jnp.max(hk, axis=0, keepdims=True),
          jnp.mean(hk, axis=0, keepdims=True),
      ],
      axis=1,
  )
  return hk, A2, x1


def _stage2(g2, A2, x1, lt2):
  f32 = jnp.float32
  hp = lax.Precision.HIGHEST

  rs2 = jnp.sum(A2, axis=1, keepdims=True)
  dinv2 = lax.rsqrt(rs2 + 1.0)
  gs = dinv2 * g2
  u2 = jnp.dot(A2, gs, preferred_element_type=f32, precision=hp) + gs
  h2 = jnp.maximum(dinv2 * u2, 0.0)

  agg2 = jnp.dot(A2, h2, preferred_element_type=f32, precision=hp) / (
      rs2 + 1e-10
  )
  score2 = jnp.sum(jnp.abs(h2 - agg2), axis=1, keepdims=True)

  S2T = _topk_select(score2, K1, K2, *lt2)
  h3 = lax.dot_general(
      S2T, h2, (((0,), (0,)), ((), ())), preferred_element_type=f32,
      precision=hp,
  )

  x2 = jnp.concatenate(
      [
          jnp.max(h3, axis=0, keepdims=True),
          jnp.mean(h3, axis=0, keepdims=True),
      ],
      axis=1,
  )

  return h3, jnp.maximum(x1, 0.0) + jnp.maximum(x2, 0.0)


def _head_body(
    xf_ref, x3_ref, w1a_ref, w1b_ref, b1_ref, g_ref, bb_ref, w3_ref, b3_ref,
    out_ref,
):
  f32 = jnp.float32
  xf = jnp.maximum(xf_ref[...], 0.0)
  xc = jnp.concatenate([xf, x3_ref[...]], axis=1)
  w1 = jnp.concatenate([w1a_ref[...], w1b_ref[...]], axis=0)
  pre = jnp.dot(xc, w1, preferred_element_type=f32) + b1_ref[...]
  feats = jnp.maximum(pre, 0.0)
  mu = jnp.mean(feats, axis=0, keepdims=True)
  var = jnp.mean((feats - mu) ** 2, axis=0, keepdims=True)
  normed = (feats - mu) * lax.rsqrt(var + 1e-5) * g_ref[...] + bb_ref[...]
  out_ref[...] = (
      jnp.dot(normed, w3_ref[...], preferred_element_type=f32) + b3_ref[...]
  )


def kernel(x, edge_index, W1, W2, lin1_w, lin1_b, bn_g, bn_b, lin3_w, lin3_b):
  off = (jnp.arange(B, dtype=jnp.int32) * NPG)[:, None]
  srcl = edge_index[0].reshape(B, EPG) - off
  dstl = edge_index[1].reshape(B, EPG) - off

  A = pl.kernel(
      _adj_body,
      out_type=jax.ShapeDtypeStruct((B, NPG, AP), jnp.float32),
      mesh=plsc.VectorSubcoreMesh(core_axis_name="c", subcore_axis_name="s"),
      scratch_types=[
          pltpu.VMEM((EPG,), jnp.int32),
          pltpu.VMEM((EPG,), jnp.int32),
          pltpu.VMEM((NPG, AP), jnp.float32),
      ],
      compiler_params=pltpu.CompilerParams(needs_layout_passes=False),
  )(srcl, dstl)

  xg = x.reshape(B, NPG, D1)
  h3, x3 = pl.pallas_call(
      _gnn_body,
      grid=(B // GPP,),
      in_specs=[
          pl.BlockSpec((GPP, NPG, D1), lambda i: (i, 0, 0)),
          pl.BlockSpec((GPP, NPG, AP), lambda i: (i, 0, 0)),
          pl.BlockSpec((1, D1, D2), lambda i: (0, 0, 0)),
          pl.BlockSpec((1, D2, D2), lambda i: (0, 0, 0)),
      ],
      out_specs=[
          pl.BlockSpec((GPP, K2, D2), lambda i: (i, 0, 0)),
          pl.BlockSpec((GPP, 1, NHID), lambda i: (i, 0, 0)),
      ],
      out_shape=[
          jax.ShapeDtypeStruct((B, K2, D2), jnp.float32),
          jax.ShapeDtypeStruct((B, 1, NHID), jnp.float32),
      ],
      compiler_params=pltpu.CompilerParams(
          dimension_semantics=("arbitrary",)
      ),
  )(xg, A, W1.reshape(1, D1, D2), W2.reshape(1, D2, D2))

  xf = h3.reshape(B, K2 * D2)
  x3 = x3.reshape(B, NHID)

  out = pl.pallas_call(
      _head_body,
      out_shape=jax.ShapeDtypeStruct((B, 128), jnp.float32),
  )(
      xf,
      x3,
      lin1_w[: K2 * D2],
      lin1_w[K2 * D2 :],
      lin1_b.reshape(1, NHID),
      bn_g.reshape(1, NHID),
      bn_b.reshape(1, NHID),
      jnp.pad(lin3_w, ((0, 0), (0, 127))),
      jnp.pad(lin3_b.reshape(1, 1), ((0, 0), (0, 127))),
  )
  return out[:, 0]
